# Initial kernel scaffold; baseline (speedup 1.0000x reference)
import jax
import jax.numpy as jnp
from jax.experimental import pallas as pl


def kernel(xyz, features, params):
    raise NotImplementedError("write your pallas kernel here")


# speedup vs baseline: 21.2455x; 21.2455x over previous
import functools

import jax
import jax.numpy as jnp
from jax import lax
from jax.experimental import pallas as pl
from jax.experimental.pallas import tpu as pltpu
from jax.experimental.pallas import tpu_sc as plsc

_LAYER_SETUP = [
    ("layer1", 2, (0.05, 0.1), (16, 32)),
    ("layer2", 1, (0.1, 0.2), (16, 32)),
    ("layer3", 0, (0.2, 0.4), (16, 32)),
    ("layer4", 0, (0.4, 0.8), (16, 32)),
]


def _round_up(x, m):
    return (x + m - 1) // m * m


def _ball_query_pallas(src, qarr, B, M, Npad, Mt, r2s, nss):
    mtiles = M // Mt

    def kern(src_ref, q_ref, o1_ref, o2_ref):
        b = pl.program_id(0)
        s = src_ref[...]
        q = q_ref[...]
        d2 = ((q[:, 0:1] - s[0:1, :]) ** 2
              + (q[:, 1:2] - s[1:2, :]) ** 2
              + (q[:, 2:3] - s[2:3, :]) ** 2)
        base = b * Npad
        for r2, ns, oref in ((r2s[0], nss[0], o1_ref), (r2s[1], nss[1], o2_ref)):
            mask = (d2 < r2).astype(jnp.float32)
            csum = mask
            sh = 1
            while sh < Npad:
                shifted = jnp.concatenate(
                    [jnp.zeros((Mt, sh), jnp.float32), csum[:, :Npad - sh]],
                    axis=1)
                csum = csum + shifted
                sh *= 2
            cols = []
            for si in range(ns):
                cols.append(jnp.sum((csum <= float(si)).astype(jnp.float32),
                                    axis=1, keepdims=True))
            idxm = jnp.concatenate(cols, axis=1)
            cnt = csum[:, Npad - 1:Npad]
            siota = lax.broadcasted_iota(jnp.int32, (Mt, ns), 1).astype(
                jnp.float32)
            idxm = jnp.where(siota < cnt, idxm, idxm[:, 0:1])
            oref[...] = idxm.astype(jnp.int32) + base

    return pl.pallas_call(
        kern,
        grid=(B, mtiles),
        in_specs=[
            pl.BlockSpec((8, Npad), lambda b, m: (b, 0)),
            pl.BlockSpec((Mt, 8), lambda b, m: (b * mtiles + m, 0)),
        ],
        out_specs=[
            pl.BlockSpec((Mt, nss[0]), lambda b, m: (b * mtiles + m, 0)),
            pl.BlockSpec((Mt, nss[1]), lambda b, m: (b * mtiles + m, 0)),
        ],
        out_shape=[
            jax.ShapeDtypeStruct((B * M, nss[0]), jnp.int32),
            jax.ShapeDtypeStruct((B * M, nss[1]), jnp.int32),
        ],
    )(src, qarr)


def _sc_gather(table, gidx):
    P = gidx.shape[0]
    D = table.shape[1]
    info = plsc.get_sparse_core_info()
    nw = info.num_cores * info.num_subcores
    bpw = P // nw
    chunk = None
    for c in range(bpw, 0, -1):
        if bpw % c == 0 and c % 8 == 0 and c * D <= 100000:
            chunk = c
            break
    nchunks = bpw // chunk
    mesh = plsc.VectorSubcoreMesh(core_axis_name="c", subcore_axis_name="s")

    @functools.partial(
        pl.kernel,
        mesh=mesh,
        out_type=jax.ShapeDtypeStruct((P, D), jnp.float32),
        scratch_types=[
            pltpu.VMEM((chunk,), jnp.int32),
            pltpu.VMEM((chunk, D), jnp.float32),
            pltpu.SemaphoreType.DMA,
        ],
    )
    def k(table_hbm, idx_hbm, out_hbm, idx_v, rows_v, sem):
        wid = lax.axis_index("s") * info.num_cores + lax.axis_index("c")
        base = wid * bpw
        for i in range(nchunks):
            off = base + i * chunk
            pltpu.sync_copy(idx_hbm.at[pl.ds(off, chunk)], idx_v)
            pltpu.async_copy(table_hbm.at[idx_v], rows_v, sem).wait()
            pltpu.sync_copy(rows_v, out_hbm.at[pl.ds(off, chunk)])

    return k(table, gidx)


def _stats_partial(y, O):
    return jnp.concatenate(
        [jnp.sum(y, axis=0, keepdims=True),
         jnp.sum(y * y, axis=0, keepdims=True),
         jnp.zeros((6, O), jnp.float32)], axis=0)


def _conv_first(x, wt, centers, ns, Mt2):
    BM = centers.shape[0]
    ntiles = BM // Mt2
    R = Mt2 * ns
    D, O = wt.shape

    def kern(x_ref, w_ref, c_ref, y_ref, st_ref):
        wv = w_ref[...]
        cpad = jnp.concatenate(
            [c_ref[...][:, 0:3], jnp.zeros((Mt2, D - 3), jnp.float32)], axis=1)
        xr = (x_ref[...].reshape(Mt2, ns, D) - cpad[:, None, :]).reshape(R, D)
        y = jnp.dot(xr, wv, preferred_element_type=jnp.float32)
        y_ref[...] = y

        @pl.when(pl.program_id(0) == 0)
        def _():
            st_ref[...] = jnp.zeros((8, O), jnp.float32)

        st_ref[...] += _stats_partial(y, O)

    return pl.pallas_call(
        kern,
        grid=(ntiles,),
        in_specs=[
            pl.BlockSpec((R, D), lambda i: (i, 0)),
            pl.BlockSpec((D, O), lambda i: (0, 0)),
            pl.BlockSpec((Mt2, 8), lambda i: (i, 0)),
        ],
        out_specs=[
            pl.BlockSpec((R, O), lambda i: (i, 0)),
            pl.BlockSpec((8, O), lambda i: (0, 0)),
        ],
        out_shape=[
            jax.ShapeDtypeStruct((BM * ns, O), jnp.float32),
            jax.ShapeDtypeStruct((8, O), jnp.float32),
        ],
    )(x, wt, centers)


def _conv_mid(x, wt, sb, Rt):
    P = x.shape[0]
    Cin, O = wt.shape
    ntiles = P // Rt

    def kern(x_ref, w_ref, sb_ref, y_ref, st_ref):
        sbv = sb_ref[...]
        xa = jnp.maximum(x_ref[...] * sbv[0:1, :] + sbv[1:2, :], 0.0)
        y = jnp.dot(xa, w_ref[...], preferred_element_type=jnp.float32)
        y_ref[...] = y

        @pl.when(pl.program_id(0) == 0)
        def _():
            st_ref[...] = jnp.zeros((8, O), jnp.float32)

        st_ref[...] += _stats_partial(y, O)

    return pl.pallas_call(
        kern,
        grid=(ntiles,),
        in_specs=[
            pl.BlockSpec((Rt, Cin), lambda i: (i, 0)),
            pl.BlockSpec((Cin, O), lambda i: (0, 0)),
            pl.BlockSpec((8, Cin), lambda i: (0, 0)),
        ],
        out_specs=[
            pl.BlockSpec((Rt, O), lambda i: (i, 0)),
            pl.BlockSpec((8, O), lambda i: (0, 0)),
        ],
        out_shape=[
            jax.ShapeDtypeStruct((P, O), jnp.float32),
            jax.ShapeDtypeStruct((8, O), jnp.float32),
        ],
    )(x, wt, sb)


def _finalize(y, sb, ns, Mt2):
    P, O = y.shape
    BM = P // ns
    ntiles = BM // Mt2
    R = Mt2 * ns

    def kern(y_ref, sb_ref, o_ref):
        sbv = sb_ref[...]
        z = jnp.maximum(y_ref[...] * sbv[0:1, :] + sbv[1:2, :], 0.0)
        o_ref[...] = jnp.max(z.reshape(Mt2, ns, O), axis=1)

    return pl.pallas_call(
        kern,
        grid=(ntiles,),
        in_specs=[
            pl.BlockSpec((R, O), lambda i: (i, 0)),
            pl.BlockSpec((8, O), lambda i: (0, 0)),
        ],
        out_specs=pl.BlockSpec((Mt2, O), lambda i: (i, 0)),
        out_shape=jax.ShapeDtypeStruct((BM, O), jnp.float32),
    )(y, sb)


def _bn_fold(stats, count, gamma, beta):
    mean = stats[0] / count
    var = stats[1] / count - mean * mean
    inv = gamma / jnp.sqrt(var + 1e-5)
    scale = inv
    shift = beta - mean * inv
    O = gamma.shape[0]
    sb = jnp.zeros((8, O), jnp.float32).at[0].set(scale).at[1].set(shift)
    return sb


def _sa_layer(xyz_map, feat_map, new_map, radii, nss, branch_params):
    B = xyz_map.shape[0]
    N = xyz_map.shape[2] * xyz_map.shape[3]
    H, W = new_map.shape[2], new_map.shape[3]
    M = H * W
    C = feat_map.shape[1]
    Npad = _round_up(N, 128)
    D = _round_up(3 + C, 128)
    Mt = 64 if M % 64 == 0 else 48
    Mt2 = 32

    xyz_rows = xyz_map.reshape(B, 3, N)
    src = jnp.pad(xyz_rows, ((0, 0), (0, 5), (0, Npad - N)),
                  constant_values=1e9).reshape(B * 8, Npad)
    new_pts = new_map.reshape(B, 3, M).transpose(0, 2, 1)
    qarr = jnp.pad(new_pts, ((0, 0), (0, 0), (0, 5))).reshape(B * M, 8)

    xyz_pts = xyz_rows.transpose(0, 2, 1)
    feat_pts = feat_map.reshape(B, C, N).transpose(0, 2, 1)
    table = jnp.concatenate([xyz_pts, feat_pts], axis=2)
    table = jnp.pad(table, ((0, 0), (0, Npad - N), (0, D - 3 - C)))
    table = table.reshape(B * Npad, D)

    r2s = tuple(r * r for r in radii)
    idx_pair = _ball_query_pallas(src, qarr, B, M, Npad, Mt, r2s, nss)

    outs = []
    for idx, ns, layers in zip(idx_pair, nss, branch_params):
        P = B * M * ns
        gath = _sc_gather(table, idx.reshape(P))

        (w1, g1, b1), (w2, g2, b2), (w3, g3, b3) = layers
        wt1 = jnp.pad(w1.T, ((0, D - w1.shape[1]), (0, 0)))
        y1, st1 = _conv_first(gath, wt1, qarr, ns, Mt2)
        sb1 = _bn_fold(st1, float(P), g1, b1)
        y2, st2 = _conv_mid(y1, w2.T, sb1, 512)
        sb2 = _bn_fold(st2, float(P), g2, b2)
        y3, st3 = _conv_mid(y2, w3.T, sb2, 512)
        sb3 = _bn_fold(st3, float(P), g3, b3)
        o = _finalize(y3, sb3, ns, Mt2)
        outs.append(o.reshape(B, M, -1))

    feat_out = jnp.concatenate(outs, axis=2).transpose(0, 2, 1)
    return feat_out.reshape(B, -1, H, W)


def kernel(xyz, features, params):
    outs = [features]
    xyz_map = xyz
    feat_map = features
    for name, n_pool, radii, nss in _LAYER_SETUP:
        new_map = xyz_map
        for _ in range(n_pool):
            new_map = new_map[:, :, ::2, ::2]
        feat_map = _sa_layer(xyz_map, feat_map, new_map, radii, nss,
                             params[name])
        xyz_map = new_map
        outs.append(feat_map)
    return tuple(outs)
